# Initial kernel scaffold; baseline (speedup 1.0000x reference)
import jax
import jax.numpy as jnp
from jax.experimental import pallas as pl


def kernel(i, j, relative_attention_bias):
    raise NotImplementedError("write your pallas kernel here")


# speedup vs baseline: 102.4140x; 102.4140x over previous
import jax
import jax.numpy as jnp
from jax.experimental import pallas as pl
from jax.experimental.pallas import tpu as pltpu

NUM_BUCKETS = 32
MAX_DISTANCE = 128
HEADS = 16
S_I = 2048
S_J = 2048

BI = 8
TLANE = 4224
_LOG_DENOM = 2.0794415416798357


def _body(delta_ref, tab_ref, out_ref, vpad_ref, t_ref):
    ib = pl.program_id(0)

    @pl.when(ib == 0)
    def _init():
        k = jax.lax.broadcasted_iota(jnp.int32, (1, TLANE), 1)
        rel = k - S_I + delta_ref[0]
        n = jnp.maximum(-rel, 0)
        max_exact = NUM_BUCKETS // 2
        is_small = n < max_exact
        safe_n = jnp.maximum(n, 1)
        val_if_large = max_exact + (
            jnp.log(safe_n.astype(jnp.float32) / max_exact)
            / _LOG_DENOM
            * (NUM_BUCKETS - max_exact)
        ).astype(jnp.int32)
        val_if_large = jnp.minimum(val_if_large, NUM_BUCKETS - 1)
        bucket = jnp.where(is_small, n, val_if_large)
        rows = jax.lax.broadcasted_iota(jnp.int32, (NUM_BUCKETS, TLANE), 0)
        onehot = (rows == bucket).astype(jnp.float32)
        vpad = jnp.dot(tab_ref[:, :], onehot, preferred_element_type=jnp.float32)
        vpad_ref[:, :] = vpad
        for t in range(128):
            t_ref[:, t, :] = pltpu.roll(vpad_ref[:, :], t, axis=1)

    i0 = ib * BI
    s0 = (S_I - 1) - i0
    b0 = jax.lax.rem(s0, 128)
    q0 = jax.lax.div(s0, 128)
    t0 = jax.lax.div(127 - b0, 8) * 8
    a0 = (q0 + 1) * 128
    out_ref[:, :, :] = t_ref[:, pl.ds(t0, BI), pl.ds(a0, S_J)]


def kernel(i, j, relative_attention_bias):
    delta = (jnp.asarray(j, jnp.int32) - S_J) - (jnp.asarray(i, jnp.int32) - S_I)
    delta = delta.reshape((1,))
    tab_t = relative_attention_bias.T
    return pl.pallas_call(
        _body,
        grid=(S_I // BI,),
        in_specs=[
            pl.BlockSpec(memory_space=pltpu.SMEM),
            pl.BlockSpec((HEADS, NUM_BUCKETS), lambda ib: (0, 0)),
        ],
        out_specs=pl.BlockSpec((HEADS, BI, S_J), lambda ib: (0, ib, 0)),
        out_shape=jax.ShapeDtypeStruct((HEADS, S_I, S_J), jnp.float32),
        scratch_shapes=[
            pltpu.VMEM((HEADS, TLANE), jnp.float32),
            pltpu.VMEM((HEADS, 128, TLANE), jnp.float32),
        ],
    )(delta, tab_t)


# speedup vs baseline: 109.5647x; 1.0698x over previous
import jax
import jax.numpy as jnp
from jax.experimental import pallas as pl
from jax.experimental.pallas import tpu as pltpu

NUM_BUCKETS = 32
MAX_DISTANCE = 128
HEADS = 16
S_I = 2048
S_J = 2048

BI = 8
TLANE = 4224
_LOG_DENOM = 2.0794415416798357


def _body(delta_ref, tab_ref, out_ref, b_ref, t_ref):
    ib = pl.program_id(0)

    @pl.when(ib == 0)
    def _init():
        k = jax.lax.broadcasted_iota(jnp.int32, (1, TLANE), 1)
        rel = k - S_I + delta_ref[0]
        n = jnp.maximum(-rel, 0)
        max_exact = NUM_BUCKETS // 2
        is_small = n < max_exact
        safe_n = jnp.maximum(n, 1)
        val_if_large = max_exact + (
            jnp.log(safe_n.astype(jnp.float32) / max_exact)
            / _LOG_DENOM
            * (NUM_BUCKETS - max_exact)
        ).astype(jnp.int32)
        val_if_large = jnp.minimum(val_if_large, NUM_BUCKETS - 1)
        bucket = jnp.where(is_small, n, val_if_large)
        rows = jax.lax.broadcasted_iota(jnp.int32, (NUM_BUCKETS, TLANE), 0)
        onehot = (rows == bucket).astype(jnp.float32)
        vpad = jnp.dot(tab_ref[:, :], onehot, preferred_element_type=jnp.float32)
        for u in range(BI):
            b_ref[:, u, :] = pltpu.roll(vpad, u, axis=1)

    for m in range(16):
        @pl.when(ib == m)
        def _build_band(m=m):
            t_ref[:, BI * m:BI * (m + 1), :] = pltpu.roll(
                b_ref[:, :, :], BI * m, axis=2
            )

    i0 = ib * BI
    s0 = (S_I - 1) - i0
    b0 = jax.lax.rem(s0, 128)
    q0 = jax.lax.div(s0, 128)
    t0 = jax.lax.div(127 - b0, 8) * 8
    a0 = (q0 + 1) * 128
    out_ref[:, :, :] = t_ref[:, pl.ds(t0, BI), pl.ds(a0, S_J)]


def kernel(i, j, relative_attention_bias):
    delta = (jnp.asarray(j, jnp.int32) - S_J) - (jnp.asarray(i, jnp.int32) - S_I)
    delta = delta.reshape((1,))
    tab_t = relative_attention_bias.T
    return pl.pallas_call(
        _body,
        grid=(S_I // BI,),
        in_specs=[
            pl.BlockSpec(memory_space=pltpu.SMEM),
            pl.BlockSpec((HEADS, NUM_BUCKETS), lambda ib: (0, 0)),
        ],
        out_specs=pl.BlockSpec((HEADS, BI, S_J), lambda ib: (0, ib, 0)),
        out_shape=jax.ShapeDtypeStruct((HEADS, S_I, S_J), jnp.float32),
        scratch_shapes=[
            pltpu.VMEM((HEADS, BI, TLANE), jnp.float32),
            pltpu.VMEM((HEADS, 128, TLANE), jnp.float32),
        ],
    )(delta, tab_t)


# speedup vs baseline: 182.5354x; 1.6660x over previous
import jax
import jax.numpy as jnp
from jax.experimental import pallas as pl
from jax.experimental.pallas import tpu as pltpu

NUM_BUCKETS = 32
MAX_DISTANCE = 128
HEADS = 16
S_I = 2048
S_J = 2048

BI = 8
TLANE = 4224
_LOG_DENOM = 2.0794415416798357


NBUF = 4


def _body(delta_ref, tab_ref, out_ref, b_ref, t_ref, sem):
    ib = pl.program_id(0)

    @pl.when(ib == 0)
    def _init():
        k = jax.lax.broadcasted_iota(jnp.int32, (1, TLANE), 1)
        rel = k - S_I + delta_ref[0]
        n = jnp.maximum(-rel, 0)
        max_exact = NUM_BUCKETS // 2
        is_small = n < max_exact
        safe_n = jnp.maximum(n, 1)
        val_if_large = max_exact + (
            jnp.log(safe_n.astype(jnp.float32) / max_exact)
            / _LOG_DENOM
            * (NUM_BUCKETS - max_exact)
        ).astype(jnp.int32)
        val_if_large = jnp.minimum(val_if_large, NUM_BUCKETS - 1)
        bucket = jnp.where(is_small, n, val_if_large)
        rows = jax.lax.broadcasted_iota(jnp.int32, (NUM_BUCKETS, TLANE), 0)
        onehot = (rows == bucket).astype(jnp.float32)
        vpad = jnp.dot(tab_ref[:, :], onehot, preferred_element_type=jnp.float32)
        for u in range(BI):
            b_ref[:, u, :] = pltpu.roll(vpad, u, axis=1)

    for m in range(16):
        @pl.when(ib == m)
        def _build_band(m=m):
            t_ref[:, BI * m:BI * (m + 1), :] = pltpu.roll(
                b_ref[:, :, :], BI * m, axis=2
            )

    i0 = ib * BI
    s0 = (S_I - 1) - i0
    b0 = jax.lax.rem(s0, 128)
    q0 = jax.lax.div(s0, 128)
    t0 = jax.lax.div(127 - b0, 8) * 8
    a0 = (q0 + 1) * 128

    def _copy(step):
        src_t0 = jax.lax.div(127 - jax.lax.rem((S_I - 1) - step * BI, 128), 8) * 8
        src_a0 = (jax.lax.div((S_I - 1) - step * BI, 128) + 1) * 128
        return pltpu.make_async_copy(
            t_ref.at[:, pl.ds(src_t0, BI), pl.ds(src_a0, S_J)],
            out_ref.at[:, pl.ds(step * BI, BI), :],
            sem,
        )

    pltpu.make_async_copy(
        t_ref.at[:, pl.ds(t0, BI), pl.ds(a0, S_J)],
        out_ref.at[:, pl.ds(i0, BI), :],
        sem,
    ).start()

    @pl.when(ib >= NBUF - 1)
    def _drain_one():
        _copy(ib - (NBUF - 1)).wait()

    @pl.when(ib == pl.num_programs(0) - 1)
    def _drain_rest():
        for lag in range(NBUF - 2, -1, -1):
            _copy(ib - lag).wait()


def kernel(i, j, relative_attention_bias):
    delta = (jnp.asarray(j, jnp.int32) - S_J) - (jnp.asarray(i, jnp.int32) - S_I)
    delta = delta.reshape((1,))
    tab_t = relative_attention_bias.T
    return pl.pallas_call(
        _body,
        grid=(S_I // BI,),
        in_specs=[
            pl.BlockSpec(memory_space=pltpu.SMEM),
            pl.BlockSpec((HEADS, NUM_BUCKETS), lambda ib: (0, 0)),
        ],
        out_specs=pl.BlockSpec(memory_space=pltpu.HBM),
        out_shape=jax.ShapeDtypeStruct((HEADS, S_I, S_J), jnp.float32),
        scratch_shapes=[
            pltpu.VMEM((HEADS, BI, TLANE), jnp.float32),
            pltpu.VMEM((HEADS, 128, TLANE), jnp.float32),
            pltpu.SemaphoreType.DMA,
        ],
    )(delta, tab_t)
